# baseline (speedup 1.0000x reference)
import functools

import jax
import jax.numpy as jnp
from jax import lax
from jax.experimental import pallas as pl
from jax.experimental.pallas import tpu as pltpu
from jax.experimental.pallas import tpu_sc as plsc

_R = 131072
_NC = 64
_NB = 129
_NK = _NC + 1
_OB = _NC + _NB - 1
_ROWS = 194
_PAD = 0.01
_EPS = 1e-5
_NCORES = 2
_NSUB = 16
_NW = _NCORES * _NSUB
_L = 16
_NG = 2
_G = _NG * _L
_RPW = _R // _NW
_CHUNKS = _RPW // _G
_OSZ = _G * _OB


def _sc_sample(wf, bf, mb1d, zf_hbm):
    mesh = plsc.VectorSubcoreMesh(core_axis_name="c", subcore_axis_name="s")

    @functools.partial(
        pl.kernel,
        out_type=[
            jax.ShapeDtypeStruct((_R * _OB,), jnp.float32),
            jax.ShapeDtypeStruct((_R,), jnp.float32),
        ],
        mesh=mesh,
        compiler_params=pltpu.CompilerParams(needs_layout_passes=False),
        scratch_types=[
            pltpu.VMEM((_G * _NC,), jnp.float32),
            pltpu.VMEM((_G * _NC,), jnp.float32),
            pltpu.VMEM((_G * _NC,), jnp.float32),
            pltpu.VMEM((_G * _NC,), jnp.float32),
            pltpu.VMEM((_G,), jnp.float32),
            pltpu.VMEM((_G,), jnp.float32),
            pltpu.VMEM((_ROWS * _G,), jnp.float32),
            pltpu.VMEM((_ROWS * _G,), jnp.float32),
            pltpu.VMEM((_ROWS * _G,), jnp.float32),
            pltpu.VMEM((_ROWS * _G,), jnp.float32),
            pltpu.VMEM((_OSZ,), jnp.float32),
            pltpu.VMEM((_OSZ,), jnp.float32),
            pltpu.VMEM((_G,), jnp.float32),
            pltpu.VMEM((_G,), jnp.float32),
            pltpu.SemaphoreType.DMA,
            pltpu.SemaphoreType.DMA,
            pltpu.SemaphoreType.DMA,
            pltpu.SemaphoreType.DMA,
            pltpu.SemaphoreType.DMA,
            pltpu.SemaphoreType.DMA,
        ],
    )
    def kern(w_hbm, b_hbm, mb_hbm, z_hbm, ob_hbm, omb_hbm,
             wbuf0, wbuf1, bbuf0, bbuf1, mbuf0, mbuf1,
             dV0, dV1, dB0, dB1, obuf0, obuf1, mstage0, mstage1,
             isem0, isem1, zsem0, zsem1, osem0, osem1):
        wbufs = (wbuf0, wbuf1)
        bbufs = (bbuf0, bbuf1)
        mbufs = (mbuf0, mbuf1)
        dVs = (dV0, dV1)
        dBs = (dB0, dB1)
        obufs = (obuf0, obuf1)
        mstages = (mstage0, mstage1)
        isems = (isem0, isem1)
        zsems = (zsem0, zsem1)
        osems = (osem0, osem1)

        wid = lax.axis_index("s") * _NCORES + lax.axis_index("c")
        base0 = wid * _RPW
        lane = lax.iota(jnp.int32, _L)
        zf = jnp.zeros((_L,), jnp.float32)
        zi = jnp.zeros((_L,), jnp.int32)
        oi = jnp.ones((_L,), jnp.int32)
        glv = [lane + g * _L for g in range(_NG)]
        obase = [(jnp.asarray(g * _L, jnp.int32) + lane) * _OB
                 for g in range(_NG)]

        def in_start(p, c):
            base = base0 + c * _G
            pltpu.make_async_copy(
                w_hbm.at[pl.ds(base * _NC, _G * _NC)], wbufs[p],
                isems[p]).start()
            pltpu.make_async_copy(
                b_hbm.at[pl.ds(base * _NC, _G * _NC)], bbufs[p],
                isems[p]).start()
            pltpu.make_async_copy(
                mb_hbm.at[pl.ds(base, _G)], mbufs[p], isems[p]).start()

        def in_wait(p):
            pltpu.make_async_copy(
                w_hbm.at[pl.ds(0, _G * _NC)], wbufs[p], isems[p]).wait()
            pltpu.make_async_copy(
                b_hbm.at[pl.ds(0, _G * _NC)], bbufs[p], isems[p]).wait()
            pltpu.make_async_copy(
                mb_hbm.at[pl.ds(0, _G)], mbufs[p], isems[p]).wait()

        def z_start(p):
            pltpu.make_async_copy(z_hbm, dVs[p], zsems[p]).start()
            pltpu.make_async_copy(z_hbm, dBs[p], zsems[p]).start()

        def z_wait(p):
            pltpu.make_async_copy(z_hbm, dVs[p], zsems[p]).wait()
            pltpu.make_async_copy(z_hbm, dBs[p], zsems[p]).wait()

        def out_start(p, c):
            base = base0 + c * _G
            pltpu.make_async_copy(
                obufs[p], ob_hbm.at[pl.ds(base * _OB, _OSZ)],
                osems[p]).start()
            pltpu.make_async_copy(
                mstages[p], omb_hbm.at[pl.ds(base, _G)], osems[p]).start()

        def out_wait(p):
            pltpu.make_async_copy(
                obufs[p], ob_hbm.at[pl.ds(0, _OSZ)], osems[p]).wait()
            pltpu.make_async_copy(
                mstages[p], omb_hbm.at[pl.ds(0, _G)], osems[p]).wait()

        for p in range(2):
            in_start(p, p)
            z_start(p)

        def chunk(p, c0, c):
            wbuf, bbuf, mbuf = wbufs[p], bbufs[p], mbufs[p]
            dV, dB = dVs[p], dBs[p]
            obuf, mstage = obufs[p], mstages[p]
            in_wait(p)

            @plsc.parallel_loop(0, _NC, carry=(zf,) * _NG)
            def S(k, ss):
                return tuple(
                    ss[g] + plsc.load_gather(wbuf, [glv[g] * _NC + k])
                    for g in range(_NG))
            rinv, padc = [], []
            for g in range(_NG):
                Sg = S[g] + (_NC * _PAD)
                padding = jnp.maximum(_EPS - Sg, 0.0)
                rinv.append(1.0 / (Sg + padding))
                padc.append(padding * (1.0 / _NC) + _PAD)

            z_wait(p)

            mvec = [mbuf[pl.ds(g * _L, _L)] for g in range(_NG)]
            e0 = [plsc.load_gather(bbuf, [glv[g] * _NC]) for g in range(_NG)]
            for g in range(_NG):
                plsc.store_scatter(dV, [glv[g]], e0[g])

            def seg_step(k, ek, st, g):
                cs, cdfp, ep, tpf, idxp = st
                wk = plsc.load_gather(wbuf, [glv[g] * _NC + (k - 1)])
                cs = cs + (wk + padc[g]) * rinv[g]
                cdfk = jnp.minimum(cs, 1.0)
                y = cdfk * _NB - 0.5
                iy = y.astype(jnp.int32)
                ck = jnp.maximum(
                    iy + jnp.where(iy.astype(jnp.float32) < y, oi, zi), 0)
                d = cdfk - cdfp
                inv = jnp.where(d > 0, 1.0 / d, 0.0)
                gr = (ek - ep) * inv
                beta = gr * (1.0 / _NB)
                Am = ep + gr * ((0.5 - k) * (1.0 / _NB) - cdfp)
                dv0 = Am + beta * tpf - ep
                idx0 = idxp + _G
                plsc.addupdate_scatter(dV, [idx0], dv0)
                plsc.addupdate_scatter(dB, [idx0], beta)
                tE = ck + k
                tEf = tE.astype(jnp.float32)
                dvE = ek - (Am + beta * (tEf - 1.0))
                idxE = tE * _G + glv[g]
                plsc.addupdate_scatter(dV, [idxE], dvE)
                plsc.addupdate_scatter(dB, [idxE], -beta)
                return (cs, cdfk, ek, tEf, idxE)

            sts0 = tuple((zf, zf, e0[g], zf, glv[g]) for g in range(_NG))

            @plsc.parallel_loop(1, _NC, carry=sts0)
            def sts(k, st):
                return tuple(
                    seg_step(k, plsc.load_gather(bbuf, [glv[g] * _NC + k]),
                             st[g], g)
                    for g in range(_NG))
            for g in range(_NG):
                seg_step(_NC, mvec[g], sts[g], g)

            @pl.when(c0 < _CHUNKS - 2)
            def _():
                in_start(p, c + 2)

            @pl.when(c0 > 0)
            def _():
                out_wait(p)

            st0 = tuple((zf, zf, obase[g]) for g in range(_NG))

            @plsc.parallel_loop(0, _OB, carry=st0)
            def st(t, stc):
                off = t * _G
                out = []
                for g in range(_NG):
                    cB, val, oidx = stc[g]
                    a = dV[pl.ds(off + g * _L, _L)]
                    b = dB[pl.ds(off + g * _L, _L)]
                    cB = cB + b
                    val = val + cB + a
                    plsc.store_scatter(obuf, [oidx], val)
                    out.append((cB, val, oidx + 1))
                return tuple(out)
            off = _OB * _G
            for g in range(_NG):
                cB, val, _ = st[g]
                a = dV[pl.ds(off + g * _L, _L)]
                b = dB[pl.ds(off + g * _L, _L)]
                mstage[pl.ds(g * _L, _L)] = val + (cB + b) + a

            out_start(p, c)

            @pl.when(c0 < _CHUNKS - 2)
            def _():
                z_start(p)

        def pair(i, carry):
            c0 = i * 2
            for p in range(2):
                chunk(p, c0, c0 + p)
            return carry

        lax.fori_loop(0, _CHUNKS // 2, pair, 0)
        out_wait(0)
        out_wait(1)

    return kern(wf, bf, mb1d, zf_hbm)


def kernel(origins, directions, weights, bins, max_bin):
    ob, omb = _sc_sample(weights.reshape(_R * _NC),
                         bins.reshape(_R * _NC), max_bin[:, 0],
                         jnp.zeros((_ROWS * _G,), jnp.float32))
    return (origins, directions, ob.reshape(_R, _OB, 1), omb[:, None])


# speedup vs baseline: 1.0052x; 1.0052x over previous
import functools

import jax
import jax.numpy as jnp
from jax import lax
from jax.experimental import pallas as pl
from jax.experimental.pallas import tpu as pltpu
from jax.experimental.pallas import tpu_sc as plsc

_R = 131072
_NC = 64
_NB = 129
_NK = _NC + 1
_OB = _NC + _NB - 1
_ROWS = 194
_PAD = 0.01
_EPS = 1e-5
_NCORES = 2
_NSUB = 16
_NW = _NCORES * _NSUB
_L = 16
_NG = 4
_G = _NG * _L
_RPW = _R // _NW
_CHUNKS = _RPW // _G
_OSZ = _G * _OB


def _sc_sample(wf, bf, mb1d, zf_hbm):
    mesh = plsc.VectorSubcoreMesh(core_axis_name="c", subcore_axis_name="s")

    @functools.partial(
        pl.kernel,
        out_type=[
            jax.ShapeDtypeStruct((_R * _OB,), jnp.float32),
            jax.ShapeDtypeStruct((_R,), jnp.float32),
        ],
        mesh=mesh,
        compiler_params=pltpu.CompilerParams(needs_layout_passes=False),
        scratch_types=[
            pltpu.VMEM((_G * _NC,), jnp.float32),
            pltpu.VMEM((_G * _NC,), jnp.float32),
            pltpu.VMEM((_G * _NC,), jnp.float32),
            pltpu.VMEM((_G * _NC,), jnp.float32),
            pltpu.VMEM((_G,), jnp.float32),
            pltpu.VMEM((_G,), jnp.float32),
            pltpu.VMEM((_ROWS * _G,), jnp.float32),
            pltpu.VMEM((_ROWS * _G,), jnp.float32),
            pltpu.VMEM((_ROWS * _G,), jnp.float32),
            pltpu.VMEM((_ROWS * _G,), jnp.float32),
            pltpu.VMEM((_OSZ,), jnp.float32),
            pltpu.VMEM((_OSZ,), jnp.float32),
            pltpu.VMEM((_G,), jnp.float32),
            pltpu.VMEM((_G,), jnp.float32),
            pltpu.SemaphoreType.DMA,
            pltpu.SemaphoreType.DMA,
            pltpu.SemaphoreType.DMA,
            pltpu.SemaphoreType.DMA,
            pltpu.SemaphoreType.DMA,
            pltpu.SemaphoreType.DMA,
        ],
    )
    def kern(w_hbm, b_hbm, mb_hbm, z_hbm, ob_hbm, omb_hbm,
             wbuf0, wbuf1, bbuf0, bbuf1, mbuf0, mbuf1,
             dV0, dV1, dB0, dB1, obuf0, obuf1, mstage0, mstage1,
             isem0, isem1, zsem0, zsem1, osem0, osem1):
        wbufs = (wbuf0, wbuf1)
        bbufs = (bbuf0, bbuf1)
        mbufs = (mbuf0, mbuf1)
        dVs = (dV0, dV1)
        dBs = (dB0, dB1)
        obufs = (obuf0, obuf1)
        mstages = (mstage0, mstage1)
        isems = (isem0, isem1)
        zsems = (zsem0, zsem1)
        osems = (osem0, osem1)

        wid = lax.axis_index("s") * _NCORES + lax.axis_index("c")
        base0 = wid * _RPW
        lane = lax.iota(jnp.int32, _L)
        zf = jnp.zeros((_L,), jnp.float32)
        zi = jnp.zeros((_L,), jnp.int32)
        oi = jnp.ones((_L,), jnp.int32)
        glv = [lane + g * _L for g in range(_NG)]
        obase = [(jnp.asarray(g * _L, jnp.int32) + lane) * _OB
                 for g in range(_NG)]

        def in_start(p, c):
            base = base0 + c * _G
            pltpu.make_async_copy(
                w_hbm.at[pl.ds(base * _NC, _G * _NC)], wbufs[p],
                isems[p]).start()
            pltpu.make_async_copy(
                b_hbm.at[pl.ds(base * _NC, _G * _NC)], bbufs[p],
                isems[p]).start()
            pltpu.make_async_copy(
                mb_hbm.at[pl.ds(base, _G)], mbufs[p], isems[p]).start()

        def in_wait(p):
            pltpu.make_async_copy(
                w_hbm.at[pl.ds(0, _G * _NC)], wbufs[p], isems[p]).wait()
            pltpu.make_async_copy(
                b_hbm.at[pl.ds(0, _G * _NC)], bbufs[p], isems[p]).wait()
            pltpu.make_async_copy(
                mb_hbm.at[pl.ds(0, _G)], mbufs[p], isems[p]).wait()

        def z_start(p):
            pltpu.make_async_copy(z_hbm, dVs[p], zsems[p]).start()
            pltpu.make_async_copy(z_hbm, dBs[p], zsems[p]).start()

        def z_wait(p):
            pltpu.make_async_copy(z_hbm, dVs[p], zsems[p]).wait()
            pltpu.make_async_copy(z_hbm, dBs[p], zsems[p]).wait()

        def out_start(p, c):
            base = base0 + c * _G
            pltpu.make_async_copy(
                obufs[p], ob_hbm.at[pl.ds(base * _OB, _OSZ)],
                osems[p]).start()
            pltpu.make_async_copy(
                mstages[p], omb_hbm.at[pl.ds(base, _G)], osems[p]).start()

        def out_wait(p):
            pltpu.make_async_copy(
                obufs[p], ob_hbm.at[pl.ds(0, _OSZ)], osems[p]).wait()
            pltpu.make_async_copy(
                mstages[p], omb_hbm.at[pl.ds(0, _G)], osems[p]).wait()

        for p in range(2):
            in_start(p, p)
            z_start(p)

        def chunk(p, c0, c):
            wbuf, bbuf, mbuf = wbufs[p], bbufs[p], mbufs[p]
            dV, dB = dVs[p], dBs[p]
            obuf, mstage = obufs[p], mstages[p]
            in_wait(p)

            @plsc.parallel_loop(0, _NC, carry=(zf,) * _NG)
            def S(k, ss):
                return tuple(
                    ss[g] + plsc.load_gather(wbuf, [glv[g] * _NC + k])
                    for g in range(_NG))
            rinv, padc = [], []
            for g in range(_NG):
                Sg = S[g] + (_NC * _PAD)
                padding = jnp.maximum(_EPS - Sg, 0.0)
                rinv.append(1.0 / (Sg + padding))
                padc.append(padding * (1.0 / _NC) + _PAD)

            z_wait(p)

            mvec = [mbuf[pl.ds(g * _L, _L)] for g in range(_NG)]
            e0 = [plsc.load_gather(bbuf, [glv[g] * _NC]) for g in range(_NG)]
            for g in range(_NG):
                plsc.store_scatter(dV, [glv[g]], e0[g])

            def seg_step(k, ek, st, g):
                cs, cdfp, ep, tpf, idxp = st
                wk = plsc.load_gather(wbuf, [glv[g] * _NC + (k - 1)])
                cs = cs + (wk + padc[g]) * rinv[g]
                cdfk = jnp.minimum(cs, 1.0)
                y = cdfk * _NB - 0.5
                iy = y.astype(jnp.int32)
                ck = jnp.maximum(
                    iy + jnp.where(iy.astype(jnp.float32) < y, oi, zi), 0)
                d = cdfk - cdfp
                inv = jnp.where(d > 0, 1.0 / d, 0.0)
                gr = (ek - ep) * inv
                beta = gr * (1.0 / _NB)
                Am = ep + gr * ((0.5 - k) * (1.0 / _NB) - cdfp)
                dv0 = Am + beta * tpf - ep
                idx0 = idxp + _G
                plsc.addupdate_scatter(dV, [idx0], dv0)
                plsc.addupdate_scatter(dB, [idx0], beta)
                tE = ck + k
                tEf = tE.astype(jnp.float32)
                dvE = ek - (Am + beta * (tEf - 1.0))
                idxE = tE * _G + glv[g]
                plsc.addupdate_scatter(dV, [idxE], dvE)
                plsc.addupdate_scatter(dB, [idxE], -beta)
                return (cs, cdfk, ek, tEf, idxE)

            sts0 = tuple((zf, zf, e0[g], zf, glv[g]) for g in range(_NG))

            @plsc.parallel_loop(1, _NC, carry=sts0)
            def sts(k, st):
                return tuple(
                    seg_step(k, plsc.load_gather(bbuf, [glv[g] * _NC + k]),
                             st[g], g)
                    for g in range(_NG))
            for g in range(_NG):
                seg_step(_NC, mvec[g], sts[g], g)

            @pl.when(c0 < _CHUNKS - 2)
            def _():
                in_start(p, c + 2)

            @pl.when(c0 > 0)
            def _():
                out_wait(p)

            st0 = tuple((zf, zf, obase[g]) for g in range(_NG))

            @plsc.parallel_loop(0, _OB, carry=st0)
            def st(t, stc):
                off = t * _G
                out = []
                for g in range(_NG):
                    cB, val, oidx = stc[g]
                    a = dV[pl.ds(off + g * _L, _L)]
                    b = dB[pl.ds(off + g * _L, _L)]
                    cB = cB + b
                    val = val + cB + a
                    plsc.store_scatter(obuf, [oidx], val)
                    out.append((cB, val, oidx + 1))
                return tuple(out)
            off = _OB * _G
            for g in range(_NG):
                cB, val, _ = st[g]
                a = dV[pl.ds(off + g * _L, _L)]
                b = dB[pl.ds(off + g * _L, _L)]
                mstage[pl.ds(g * _L, _L)] = val + (cB + b) + a

            out_start(p, c)

            @pl.when(c0 < _CHUNKS - 2)
            def _():
                z_start(p)

        def pair(i, carry):
            c0 = i * 2
            for p in range(2):
                chunk(p, c0, c0 + p)
            return carry

        lax.fori_loop(0, _CHUNKS // 2, pair, 0)
        out_wait(0)
        out_wait(1)

    return kern(wf, bf, mb1d, zf_hbm)


def kernel(origins, directions, weights, bins, max_bin):
    ob, omb = _sc_sample(weights.reshape(_R * _NC),
                         bins.reshape(_R * _NC), max_bin[:, 0],
                         jnp.zeros((_ROWS * _G,), jnp.float32))
    return (origins, directions, ob.reshape(_R, _OB, 1), omb[:, None])
